# Initial kernel scaffold; baseline (speedup 1.0000x reference)
import functools

import jax
import jax.numpy as jnp
from jax import lax
from jax.experimental import pallas as pl
from jax.experimental.pallas import tpu as pltpu
from jax.experimental.pallas import tpu_sc as plsc

EMB = 32
G = 128
K = 8
SUPER = G * K
NC, NS = 2, 16
NW = NC * NS


def _sc_dual_gather_add(n_groups: int):
    groups_per_w = n_groups // NW
    n_super = groups_per_w // K
    out_rows = n_groups * G

    mesh = plsc.VectorSubcoreMesh(core_axis_name="c", subcore_axis_name="s")

    @functools.partial(
        pl.kernel,
        mesh=mesh,
        out_type=jax.ShapeDtypeStruct((out_rows, EMB), jnp.float32),
        scratch_types=[
            pltpu.VMEM((K, G), jnp.int32),
            pltpu.VMEM((K, G), jnp.int32),
            pltpu.VMEM((SUPER, EMB), jnp.float32),
            pltpu.VMEM((SUPER, EMB), jnp.float32),
            pltpu.SemaphoreType.DMA,
        ],
    )
    def k(i1_hbm, i2_hbm, w1_hbm, w2_hbm, o_hbm, i1v, i2v, r1v, r2v, sem):
        wid = lax.axis_index("s") * NC + lax.axis_index("c")
        gbase = wid * groups_per_w

        @pl.loop(0, n_super)
        def _(g):
            gb = gbase + g * K
            pltpu.sync_copy(i1_hbm.at[pl.ds(gb, K)], i1v)
            pltpu.sync_copy(i2_hbm.at[pl.ds(gb, K)], i2v)
            copies = []
            for j in range(K):
                dst = pl.ds(j * G, G)
                copies.append(pltpu.async_copy(w1_hbm.at[i1v.at[j]], r1v.at[dst], sem))
                copies.append(pltpu.async_copy(w2_hbm.at[i2v.at[j]], r2v.at[dst], sem))
            for c in copies:
                c.wait()

            @pl.loop(0, SUPER)
            def _(r):
                r1v[r, pl.ds(0, 16)] += r2v[r, pl.ds(0, 16)]
                r1v[r, pl.ds(16, 16)] += r2v[r, pl.ds(16, 16)]

            pltpu.sync_copy(r1v, o_hbm.at[pl.ds(gb * G, SUPER)])

    return k


def kernel(input, another_input, W1, W2):
    B, H = input.shape
    n = B * H
    idx1 = input.astype(jnp.int32).reshape(n // G, G)
    idx2 = another_input.astype(jnp.int32).reshape(n // G, G)
    out = _sc_dual_gather_add(n // G)(idx1, idx2, W1, W2)
    return out.reshape(B, H, EMB)


# speedup vs baseline: 1.3605x; 1.3605x over previous
import functools

import jax
import jax.numpy as jnp
from jax import lax
from jax.experimental import pallas as pl
from jax.experimental.pallas import tpu as pltpu
from jax.experimental.pallas import tpu_sc as plsc

EMB = 32
G = 128
K = 8
SUPER = G * K
NC, NS = 2, 16
NW = NC * NS


def _sc_dual_gather_add(n_groups: int):
    groups_per_w = n_groups // NW
    n_super = groups_per_w // K
    out_rows = n_groups * G

    mesh = plsc.VectorSubcoreMesh(core_axis_name="c", subcore_axis_name="s")

    @functools.partial(
        pl.kernel,
        mesh=mesh,
        out_type=jax.ShapeDtypeStruct((out_rows, EMB), jnp.float32),
        compiler_params=pltpu.CompilerParams(use_tc_tiling_on_sc=False),
        scratch_types=[
            pltpu.VMEM((K, G), jnp.int32),
            pltpu.VMEM((K, G), jnp.int32),
            pltpu.VMEM((SUPER, EMB), jnp.float32),
            pltpu.VMEM((SUPER, EMB), jnp.float32),
            pltpu.SemaphoreType.DMA,
        ],
    )
    def k(i1_hbm, i2_hbm, w1_hbm, w2_hbm, o_hbm, i1v, i2v, r1v, r2v, sem):
        wid = lax.axis_index("s") * NC + lax.axis_index("c")
        gbase = wid * groups_per_w

        @pl.loop(0, n_super)
        def _(g):
            gb = gbase + g * K
            pltpu.sync_copy(i1_hbm.at[pl.ds(gb, K)], i1v)
            pltpu.sync_copy(i2_hbm.at[pl.ds(gb, K)], i2v)
            copies = []
            for j in range(K):
                dst = pl.ds(j * G, G)
                copies.append(pltpu.async_copy(w1_hbm.at[i1v.at[j]], r1v.at[dst], sem))
                copies.append(pltpu.async_copy(w2_hbm.at[i2v.at[j]], r2v.at[dst], sem))
            for c in copies:
                c.wait()

            @pl.loop(0, SUPER)
            def _(r):
                r1v[r, pl.ds(0, 16)] += r2v[r, pl.ds(0, 16)]
                r1v[r, pl.ds(16, 16)] += r2v[r, pl.ds(16, 16)]

            pltpu.sync_copy(r1v, o_hbm.at[pl.ds(gb * G, SUPER)])

    return k


def kernel(input, another_input, W1, W2):
    B, H = input.shape
    n = B * H
    idx1 = input.astype(jnp.int32).reshape(n // G, G)
    idx2 = another_input.astype(jnp.int32).reshape(n // G, G)
    out = _sc_dual_gather_add(n // G)(idx1, idx2, W1, W2)
    return out.reshape(B, H, EMB)


# speedup vs baseline: 1.4284x; 1.0499x over previous
import functools

import jax
import jax.numpy as jnp
from jax import lax
from jax.experimental import pallas as pl
from jax.experimental.pallas import tpu as pltpu
from jax.experimental.pallas import tpu_sc as plsc

EMB = 32
G = 128
K = 5
SUPER = G * K
NC, NS = 2, 16
NW = NC * NS


def _sc_dual_gather_add(n_groups: int):
    groups_per_w = n_groups // NW
    n_super = groups_per_w // K
    assert n_super % 2 == 0
    out_rows = n_groups * G

    mesh = plsc.VectorSubcoreMesh(core_axis_name="c", subcore_axis_name="s")

    @functools.partial(
        pl.kernel,
        mesh=mesh,
        out_type=jax.ShapeDtypeStruct((out_rows, EMB), jnp.float32),
        compiler_params=pltpu.CompilerParams(use_tc_tiling_on_sc=False),
        scratch_types=[
            pltpu.VMEM((2, K, G), jnp.int32),
            pltpu.VMEM((2, K, G), jnp.int32),
            pltpu.VMEM((2, SUPER, EMB), jnp.float32),
            pltpu.VMEM((2, SUPER, EMB), jnp.float32),
            pltpu.SemaphoreType.DMA,
            pltpu.SemaphoreType.DMA,
            pltpu.SemaphoreType.DMA,
            pltpu.SemaphoreType.DMA,
            pltpu.SemaphoreType.DMA,
            pltpu.SemaphoreType.DMA,
        ],
    )
    def k(i1_hbm, i2_hbm, w1_hbm, w2_hbm, o_hbm,
          i1v, i2v, r1v, r2v, sg0, sg1, si0, si1, ss0, ss1):
        sem_g = (sg0, sg1)
        sem_i = (si0, si1)
        sem_s = (ss0, ss1)
        wid = lax.axis_index("s") * NC + lax.axis_index("c")
        gbase = wid * groups_per_w

        def fire_idx(g, b):
            gb = gbase + g * K
            pltpu.async_copy(i1_hbm.at[pl.ds(gb, K)], i1v.at[b], sem_i[b])
            pltpu.async_copy(i2_hbm.at[pl.ds(gb, K)], i2v.at[b], sem_i[b])

        def wait_idx(b):
            pltpu.make_async_copy(i1_hbm.at[pl.ds(0, K)], i1v.at[b], sem_i[b]).wait()
            pltpu.make_async_copy(i2_hbm.at[pl.ds(0, K)], i2v.at[b], sem_i[b]).wait()

        def fire_gathers(b):
            for j in range(K):
                dst = pl.ds(j * G, G)
                pltpu.async_copy(w1_hbm.at[i1v.at[b, j]], r1v.at[b, dst], sem_g[b])
                pltpu.async_copy(w2_hbm.at[i2v.at[b, j]], r2v.at[b, dst], sem_g[b])

        def drain_gathers(b):
            for j in range(K):
                dst = pl.ds(j * G, G)
                pltpu.make_async_copy(w1_hbm.at[i1v.at[b, j]], r1v.at[b, dst], sem_g[b]).wait()
                pltpu.make_async_copy(w2_hbm.at[i2v.at[b, j]], r2v.at[b, dst], sem_g[b]).wait()

        def fire_store(g, b):
            off = (gbase + g * K) * G
            pltpu.async_copy(r1v.at[b], o_hbm.at[pl.ds(off, SUPER)], sem_s[b])

        def wait_store(b):
            pltpu.make_async_copy(r1v.at[b], o_hbm.at[pl.ds(0, SUPER)], sem_s[b]).wait()

        def add_block(b):
            @pl.loop(0, SUPER)
            def _(r):
                r1v[b, r, pl.ds(0, 16)] += r2v[b, r, pl.ds(0, 16)]
                r1v[b, r, pl.ds(16, 16)] += r2v[b, r, pl.ds(16, 16)]

        pltpu.sync_copy(i1_hbm.at[pl.ds(gbase, K)], i1v.at[0])
        pltpu.sync_copy(i2_hbm.at[pl.ds(gbase, K)], i2v.at[0])
        fire_gathers(0)
        fire_idx(1, 1)

        @pl.loop(0, n_super // 2)
        def _(p):
            for b in range(2):
                g = 2 * p + b
                nb = 1 - b

                @pl.when(g + 1 < n_super)
                def _():
                    wait_idx(nb)

                    @pl.when(g >= 1)
                    def _():
                        wait_store(nb)

                    fire_gathers(nb)

                drain_gathers(b)
                add_block(b)
                fire_store(g, b)

                @pl.when(g + 2 < n_super)
                def _():
                    fire_idx(g + 2, b)

        wait_store(0)
        wait_store(1)

    return k


def kernel(input, another_input, W1, W2):
    B, H = input.shape
    n = B * H
    idx1 = input.astype(jnp.int32).reshape(n // G, G)
    idx2 = another_input.astype(jnp.int32).reshape(n // G, G)
    out = _sc_dual_gather_add(n // G)(idx1, idx2, W1, W2)
    return out.reshape(B, H, EMB)
